# speedup vs baseline: 1.0005x; 1.0005x over previous
import jax
import jax.numpy as jnp
from jax.experimental import pallas as pl

_LSV_INDEX = 0
_LSV_SCALING_FACTOR = 1.0
_BLOCK_ROWS = 512


def _add_row_kernel(x_ref, emb_ref, o_ref):
    row = emb_ref[_LSV_INDEX, :] * _LSV_SCALING_FACTOR
    o_ref[...] = x_ref[...] + row[None, :]


def kernel(x, lsv_embedding):
    b, s, e = x.shape
    n = b * s
    xf = x.reshape(n, e)
    out = pl.pallas_call(
        _add_row_kernel,
        grid=(n // _BLOCK_ROWS,),
        in_specs=[
            pl.BlockSpec((_BLOCK_ROWS, e), lambda i: (i, 0)),
            pl.BlockSpec(lsv_embedding.shape, lambda i: (0, 0)),
        ],
        out_specs=pl.BlockSpec((_BLOCK_ROWS, e), lambda i: (i, 0)),
        out_shape=jax.ShapeDtypeStruct((n, e), x.dtype),
    )(xf, lsv_embedding)
    return out.reshape(b, s, e)
